# baseline (speedup 1.0000x reference)
import functools

import jax
import jax.numpy as jnp
from jax import lax
from jax.experimental import pallas as pl
from jax.experimental.pallas import tpu as pltpu
from jax.experimental.pallas import tpu_sc as plsc

EMB = 16
BATCH = 16384
BN_EPS = 1e-3
NT = 32
W = BATCH // NT
WCH = W // 16


def _gather_body(idx2_hbm, uembT_hbm, ubias_hbm, bembT_hbm, bbias_hbm,
                 partials_hbm, bsum_hbm,
                 uidx_v, bidx_v, urows_v, brows_v, ub_v, bb_v, acc_v, bs_v,
                 sem):
    c = lax.axis_index("c")
    s = lax.axis_index("s")
    w = s * 2 + c
    base = w * W

    pltpu.sync_copy(idx2_hbm.at[0, pl.ds(base, W)], uidx_v)
    pltpu.sync_copy(idx2_hbm.at[1, pl.ds(base, W)], bidx_v)

    cps = []
    for e in range(EMB):
        cps.append(pltpu.async_copy(uembT_hbm.at[e].at[uidx_v],
                                    urows_v.at[e], sem))
        cps.append(pltpu.async_copy(bembT_hbm.at[e].at[bidx_v],
                                    brows_v.at[e], sem))
    cps.append(pltpu.async_copy(ubias_hbm.at[uidx_v], ub_v, sem))
    cps.append(pltpu.async_copy(bbias_hbm.at[bidx_v], bb_v, sem))
    for cp in cps:
        cp.wait()

    def mac(j, accs):
        a0, a1 = accs
        ch = pl.ds(j * 16, 16)
        for e in range(0, EMB, 2):
            a0 = a0 + urows_v[e, ch] * brows_v[e, ch]
            a1 = a1 + urows_v[e + 1, ch] * brows_v[e + 1, ch]
        bs_v[ch] = ub_v[ch] + bb_v[ch]
        return a0, a1

    z = jnp.zeros((16,), jnp.float32)
    a0, a1 = lax.fori_loop(0, WCH, mac, (z, z))
    acc_v[...] = a0 + a1

    pltpu.sync_copy(acc_v, partials_hbm.at[pl.ds(w * 16, 16)])
    pltpu.sync_copy(bs_v, bsum_hbm.at[pl.ds(base, W)])


def _finish_body(partials_hbm, bsum_hbm, params_hbm, out_hbm,
                 pt_v, bs_v, params_v, out_v, sem):
    c = lax.axis_index("c")
    s = lax.axis_index("s")
    w = s * 2 + c
    base = w * W

    pltpu.sync_copy(partials_hbm, pt_v)
    pltpu.sync_copy(bsum_hbm.at[pl.ds(base, W)], bs_v)
    pltpu.sync_copy(params_hbm, params_v)

    t = pt_v[pl.ds(0, 16)]
    for j in range(1, NT):
        t = t + pt_v[pl.ds(j * 16, 16)]
    S = t[0]
    for j in range(1, 16):
        S = S + t[j]

    scale = params_v[0, :]
    shift = params_v[1, :]

    def emit(j, carry):
        ch = pl.ds(j * 16, 16)
        x = (S + bs_v[ch]) * scale + shift
        out_v[ch] = 1.0 / (1.0 + jnp.exp(-x))
        return carry

    lax.fori_loop(0, WCH, emit, 0)
    pltpu.sync_copy(out_v, out_hbm.at[pl.ds(base, W)])


@jax.jit
def _sc_call(idx2, uembT, ubias, bembT, bbias, params):
    mesh = plsc.VectorSubcoreMesh(core_axis_name="c", subcore_axis_name="s",
                                  num_cores=2)
    k1 = pl.kernel(
        _gather_body,
        out_type=(jax.ShapeDtypeStruct((NT * 16,), jnp.float32),
                  jax.ShapeDtypeStruct((BATCH,), jnp.float32)),
        mesh=mesh,
        scratch_types=[
            pltpu.VMEM((W,), jnp.int32),
            pltpu.VMEM((W,), jnp.int32),
            pltpu.VMEM((EMB, W), jnp.float32),
            pltpu.VMEM((EMB, W), jnp.float32),
            pltpu.VMEM((W,), jnp.float32),
            pltpu.VMEM((W,), jnp.float32),
            pltpu.VMEM((16,), jnp.float32),
            pltpu.VMEM((W,), jnp.float32),
            pltpu.SemaphoreType.DMA,
        ],
        compiler_params=pltpu.CompilerParams(use_tc_tiling_on_sc=False),
    )
    partials, bsum = k1(idx2, uembT, ubias, bembT, bbias)

    k2 = pl.kernel(
        _finish_body,
        out_type=jax.ShapeDtypeStruct((BATCH,), jnp.float32),
        mesh=mesh,
        scratch_types=[
            pltpu.VMEM((NT * 16,), jnp.float32),
            pltpu.VMEM((W,), jnp.float32),
            pltpu.VMEM((2, 16), jnp.float32),
            pltpu.VMEM((W,), jnp.float32),
            pltpu.SemaphoreType.DMA,
        ],
        compiler_params=pltpu.CompilerParams(use_tc_tiling_on_sc=False),
    )
    return k2(partials, bsum, params)


def kernel(inputs, user_emb, user_bias, book_emb, book_bias,
           bn_gamma, bn_beta, bn_mean, bn_var):
    idx2 = inputs.T
    ubias = user_bias.reshape(-1)
    bbias = book_bias.reshape(-1)
    scale = bn_gamma * lax.rsqrt(bn_var + BN_EPS)
    shift = bn_beta - bn_mean * scale
    params = jnp.stack([jnp.broadcast_to(scale, (16,)),
                        jnp.broadcast_to(shift, (16,))])
    out = _sc_call(idx2, user_emb.T, ubias, book_emb.T, bbias, params)
    return out.reshape(BATCH, 1)


# speedup vs baseline: 18.4336x; 18.4336x over previous
import functools

import jax
import jax.numpy as jnp
from jax import lax
from jax.experimental import pallas as pl
from jax.experimental.pallas import tpu as pltpu
from jax.experimental.pallas import tpu_sc as plsc

EMB = 16
BATCH = 16384
BN_EPS = 1e-3
NT = 32
W = BATCH // NT
NB = 16
NWAVE = W // NB
WCH = W // 16


def _gather_body(idx2_hbm, uembT_hbm, bembT_hbm,
                 partials_hbm,
                 uidx_v, bidx_v, ublk_v, bblk_v, acc_v, sem):
    c = lax.axis_index("c")
    s = lax.axis_index("s")
    w = s * 2 + c
    base = w * W

    pltpu.sync_copy(idx2_hbm.at[0, pl.ds(base, W)], uidx_v)
    pltpu.sync_copy(idx2_hbm.at[1, pl.ds(base, W)], bidx_v)

    eiota = lax.iota(jnp.int32, 16)

    def fire(tbl, iv, blk):
        cps = []
        for u in range(NB):
            off = pl.multiple_of((iv[u] // 128) * 128, 128)
            cps.append(pltpu.async_copy(
                tbl.at[:, pl.ds(off, 128)], blk.at[u], sem))
        return cps

    def extract(iv, blk, u):
        col = jnp.full((16,), iv[u] - (iv[u] // 128) * 128, jnp.int32)
        return plsc.load_gather(blk.at[u], [eiota, col])

    def wave(j, acc):
        uiv = uidx_v[pl.ds(j * NB, NB)]
        biv = bidx_v[pl.ds(j * NB, NB)]
        ucps = fire(uembT_hbm, uiv, ublk_v)
        bcps = fire(bembT_hbm, biv, bblk_v)
        for cp in ucps:
            cp.wait()
        uvals = [extract(uiv, ublk_v, u) for u in range(NB)]
        for cp in bcps:
            cp.wait()
        for u in range(NB):
            acc = acc + uvals[u] * extract(biv, bblk_v, u)
        return acc

    acc = lax.fori_loop(0, NWAVE, wave, jnp.zeros((16,), jnp.float32))
    acc_v[...] = acc
    pltpu.sync_copy(acc_v, partials_hbm.at[pl.ds(w * 16, 16)])


def _finish_body(idx2_hbm, partials_hbm, ubias_hbm, bbias_hbm, params_hbm,
                 out_hbm,
                 uidx_v, bidx_v, pt_v, ub_v, bb_v, params_v, out_v, sem):
    c = lax.axis_index("c")
    s = lax.axis_index("s")
    w = s * 2 + c
    base = w * W

    pltpu.sync_copy(idx2_hbm.at[0, pl.ds(base, W)], uidx_v)
    pltpu.sync_copy(idx2_hbm.at[1, pl.ds(base, W)], bidx_v)
    pltpu.sync_copy(partials_hbm, pt_v)
    pltpu.sync_copy(params_hbm, params_v)

    cp1 = pltpu.async_copy(ubias_hbm.at[uidx_v], ub_v, sem)
    cp2 = pltpu.async_copy(bbias_hbm.at[bidx_v], bb_v, sem)

    t = pt_v[pl.ds(0, 16)]
    for j in range(1, NT):
        t = t + pt_v[pl.ds(j * 16, 16)]
    S = t[0]
    for j in range(1, 16):
        S = S + t[j]

    scale = params_v[0, :]
    shift = params_v[1, :]
    cp1.wait()
    cp2.wait()

    def emit(j, carry):
        ch = pl.ds(j * 16, 16)
        x = (S + ub_v[ch] + bb_v[ch]) * scale + shift
        out_v[ch] = 1.0 / (1.0 + jnp.exp(-x))
        return carry

    lax.fori_loop(0, WCH, emit, 0)
    pltpu.sync_copy(out_v, out_hbm.at[pl.ds(base, W)])


@jax.jit
def _sc_call(idx2, uembT, ubias, bembT, bbias, params):
    mesh = plsc.VectorSubcoreMesh(core_axis_name="c", subcore_axis_name="s",
                                  num_cores=2)
    k1 = pl.kernel(
        _gather_body,
        out_type=jax.ShapeDtypeStruct((NT * 16,), jnp.float32),
        mesh=mesh,
        scratch_types=[
            pltpu.VMEM((W,), jnp.int32),
            pltpu.VMEM((W,), jnp.int32),
            pltpu.VMEM((NB, 16, 128), jnp.float32),
            pltpu.VMEM((NB, 16, 128), jnp.float32),
            pltpu.VMEM((16,), jnp.float32),
            pltpu.SemaphoreType.DMA,
        ],
        compiler_params=pltpu.CompilerParams(use_tc_tiling_on_sc=True,
                                             needs_layout_passes=False),
    )
    partials = k1(idx2, uembT, bembT)

    k2 = pl.kernel(
        _finish_body,
        out_type=jax.ShapeDtypeStruct((BATCH,), jnp.float32),
        mesh=mesh,
        scratch_types=[
            pltpu.VMEM((W,), jnp.int32),
            pltpu.VMEM((W,), jnp.int32),
            pltpu.VMEM((NT * 16,), jnp.float32),
            pltpu.VMEM((W,), jnp.float32),
            pltpu.VMEM((W,), jnp.float32),
            pltpu.VMEM((2, 16), jnp.float32),
            pltpu.VMEM((W,), jnp.float32),
            pltpu.SemaphoreType.DMA,
        ],
        compiler_params=pltpu.CompilerParams(use_tc_tiling_on_sc=False),
    )
    return k2(idx2, partials, ubias, bbias, params)


def kernel(inputs, user_emb, user_bias, book_emb, book_bias,
           bn_gamma, bn_beta, bn_mean, bn_var):
    idx2 = inputs.T
    ubias = user_bias.reshape(-1)
    bbias = book_bias.reshape(-1)
    scale = bn_gamma * lax.rsqrt(bn_var + BN_EPS)
    shift = bn_beta - bn_mean * scale
    params = jnp.stack([jnp.broadcast_to(scale, (16,)),
                        jnp.broadcast_to(shift, (16,))])
    out = _sc_call(idx2, user_emb.T, ubias, book_emb.T, bbias, params)
    return out.reshape(BATCH, 1)


# speedup vs baseline: 19.2207x; 1.0427x over previous
import functools

import jax
import jax.numpy as jnp
from jax import lax
from jax.experimental import pallas as pl
from jax.experimental.pallas import tpu as pltpu
from jax.experimental.pallas import tpu_sc as plsc

EMB = 16
BATCH = 16384
BN_EPS = 1e-3
NT = 32
W = BATCH // NT
NB = 16
NWAVE = W // NB
WCH = W // 16


def _gather_body(idx2_hbm, uembT_hbm, bembT_hbm,
                 partials_hbm,
                 uidx_v, bidx_v, ua_v, ub_v, ba_v, bb_v, acc_v,
                 sua, sub, sba, sbb):
    c = lax.axis_index("c")
    s = lax.axis_index("s")
    w = s * 2 + c
    base = w * W

    pltpu.sync_copy(idx2_hbm.at[0, pl.ds(base, W)], uidx_v)
    pltpu.sync_copy(idx2_hbm.at[1, pl.ds(base, W)], bidx_v)

    eiota = lax.iota(jnp.int32, 16)

    def fire(tbl, iv, lane0, blk, sem):
        for u in range(8):
            off = pl.multiple_of((iv[lane0 + u] // 128) * 128, 128)
            pltpu.async_copy(tbl.at[:, pl.ds(off, 128)], blk.at[u], sem)

    def drain(blk, sem):
        for u in range(8):
            pltpu.make_async_copy(
                uembT_hbm.at[:, pl.ds(0, 128)], blk.at[u], sem).wait()

    def extract(iv, lane0, blk, u):
        i = iv[lane0 + u]
        col = jnp.full((16,), i - (i // 128) * 128, jnp.int32)
        return plsc.load_gather(blk.at[u], [eiota, col])

    def mac(uiv, biv, lane0, ublk, bblk, acc):
        uvals = [extract(uiv, lane0, ublk, u) for u in range(8)]
        for u in range(8):
            acc = acc + uvals[u] * extract(biv, lane0, bblk, u)
        return acc

    uiv0 = uidx_v[pl.ds(0, 16)]
    biv0 = bidx_v[pl.ds(0, 16)]
    fire(uembT_hbm, uiv0, 0, ua_v, sua)
    fire(bembT_hbm, biv0, 0, ba_v, sba)

    def step(k, carry):
        acc, uiv, biv = carry
        fire(uembT_hbm, uiv, 8, ub_v, sub)
        fire(bembT_hbm, biv, 8, bb_v, sbb)
        drain(ua_v, sua)
        drain(ba_v, sba)
        acc = mac(uiv, biv, 0, ua_v, ba_v, acc)
        nk = jnp.minimum(k + 1, NWAVE - 1)
        nuiv = uidx_v[pl.ds(nk * 16, 16)]
        nbiv = bidx_v[pl.ds(nk * 16, 16)]

        @pl.when(k + 1 < NWAVE)
        def _():
            fire(uembT_hbm, nuiv, 0, ua_v, sua)
            fire(bembT_hbm, nbiv, 0, ba_v, sba)

        drain(ub_v, sub)
        drain(bb_v, sbb)
        acc = mac(uiv, biv, 8, ub_v, bb_v, acc)
        return acc, nuiv, nbiv

    acc, _, _ = lax.fori_loop(
        0, NWAVE, step, (jnp.zeros((16,), jnp.float32), uiv0, biv0))
    acc_v[...] = acc
    pltpu.sync_copy(acc_v, partials_hbm.at[pl.ds(w * 16, 16)])


def _finish_body(idx2_hbm, partials_hbm, ubias_hbm, bbias_hbm, params_hbm,
                 out_hbm,
                 uidx_v, bidx_v, pt_v, ub_v, bb_v, params_v, out_v, sem):
    c = lax.axis_index("c")
    s = lax.axis_index("s")
    w = s * 2 + c
    base = w * W

    pltpu.sync_copy(idx2_hbm.at[0, pl.ds(base, W)], uidx_v)
    pltpu.sync_copy(idx2_hbm.at[1, pl.ds(base, W)], bidx_v)
    pltpu.sync_copy(partials_hbm, pt_v)
    pltpu.sync_copy(params_hbm, params_v)

    cp1 = pltpu.async_copy(ubias_hbm.at[uidx_v], ub_v, sem)
    cp2 = pltpu.async_copy(bbias_hbm.at[bidx_v], bb_v, sem)

    t = pt_v[pl.ds(0, 16)]
    for j in range(1, NT):
        t = t + pt_v[pl.ds(j * 16, 16)]
    S = t[0]
    for j in range(1, 16):
        S = S + t[j]

    scale = params_v[0, :]
    shift = params_v[1, :]
    cp1.wait()
    cp2.wait()

    def emit(j, carry):
        ch = pl.ds(j * 16, 16)
        x = (S + ub_v[ch] + bb_v[ch]) * scale + shift
        out_v[ch] = 1.0 / (1.0 + jnp.exp(-x))
        return carry

    lax.fori_loop(0, WCH, emit, 0)
    pltpu.sync_copy(out_v, out_hbm.at[pl.ds(base, W)])


@jax.jit
def _sc_call(idx2, uembT, ubias, bembT, bbias, params):
    mesh = plsc.VectorSubcoreMesh(core_axis_name="c", subcore_axis_name="s",
                                  num_cores=2)
    k1 = pl.kernel(
        _gather_body,
        out_type=jax.ShapeDtypeStruct((NT * 16,), jnp.float32),
        mesh=mesh,
        scratch_types=[
            pltpu.VMEM((W,), jnp.int32),
            pltpu.VMEM((W,), jnp.int32),
            pltpu.VMEM((8, 16, 128), jnp.float32),
            pltpu.VMEM((8, 16, 128), jnp.float32),
            pltpu.VMEM((8, 16, 128), jnp.float32),
            pltpu.VMEM((8, 16, 128), jnp.float32),
            pltpu.VMEM((16,), jnp.float32),
            pltpu.SemaphoreType.DMA,
            pltpu.SemaphoreType.DMA,
            pltpu.SemaphoreType.DMA,
            pltpu.SemaphoreType.DMA,
        ],
        compiler_params=pltpu.CompilerParams(use_tc_tiling_on_sc=True,
                                             needs_layout_passes=False),
    )
    partials = k1(idx2, uembT, bembT)

    k2 = pl.kernel(
        _finish_body,
        out_type=jax.ShapeDtypeStruct((BATCH,), jnp.float32),
        mesh=mesh,
        scratch_types=[
            pltpu.VMEM((W,), jnp.int32),
            pltpu.VMEM((W,), jnp.int32),
            pltpu.VMEM((NT * 16,), jnp.float32),
            pltpu.VMEM((W,), jnp.float32),
            pltpu.VMEM((W,), jnp.float32),
            pltpu.VMEM((2, 16), jnp.float32),
            pltpu.VMEM((W,), jnp.float32),
            pltpu.SemaphoreType.DMA,
        ],
        compiler_params=pltpu.CompilerParams(use_tc_tiling_on_sc=False),
    )
    return k2(idx2, partials, ubias, bbias, params)


def kernel(inputs, user_emb, user_bias, book_emb, book_bias,
           bn_gamma, bn_beta, bn_mean, bn_var):
    idx2 = inputs.T
    ubias = user_bias.reshape(-1)
    bbias = book_bias.reshape(-1)
    scale = bn_gamma * lax.rsqrt(bn_var + BN_EPS)
    shift = bn_beta - bn_mean * scale
    params = jnp.stack([jnp.broadcast_to(scale, (16,)),
                        jnp.broadcast_to(shift, (16,))])
    out = _sc_call(idx2, user_emb.T, ubias, book_emb.T, bbias, params)
    return out.reshape(BATCH, 1)


# speedup vs baseline: 19.2392x; 1.0010x over previous
import jax
import jax.numpy as jnp
from jax import lax
from jax.experimental import pallas as pl
from jax.experimental.pallas import tpu as pltpu
from jax.experimental.pallas import tpu_sc as plsc

EMB = 16
BATCH = 16384
BN_EPS = 1e-3
NT = 32
W = BATCH // NT
NB = 16
NWAVE = W // NB
WCH = W // 16


def _gather_body(idx2_hbm, uembT_hbm, bembT_hbm,
                 partials_hbm,
                 uidx_v, bidx_v, ua_v, ub_v, ba_v, bb_v, acc_v,
                 sua, sub, sba, sbb):
    c = lax.axis_index("c")
    s = lax.axis_index("s")
    w = s * 2 + c
    base = w * W

    pltpu.sync_copy(idx2_hbm.at[0, pl.ds(base, W)], uidx_v)
    pltpu.sync_copy(idx2_hbm.at[1, pl.ds(base, W)], bidx_v)

    eiota = lax.iota(jnp.int32, 16)

    def fire(tbl, iv, lane0, blk, sem):
        for u in range(8):
            off = pl.multiple_of((iv[lane0 + u] // 128) * 128, 128)
            pltpu.async_copy(tbl.at[:, pl.ds(off, 128)], blk.at[u], sem)

    def drain(blk, sem):
        for u in range(8):
            pltpu.make_async_copy(
                uembT_hbm.at[:, pl.ds(0, 128)], blk.at[u], sem).wait()

    def extract(iv, lane0, blk, u):
        i = iv[lane0 + u]
        col = jnp.full((16,), i - (i // 128) * 128, jnp.int32)
        return plsc.load_gather(blk.at[u], [eiota, col])

    def mac(uiv, biv, lane0, ublk, bblk, acc):
        uvals = [extract(uiv, lane0, ublk, u) for u in range(8)]
        for u in range(8):
            acc = acc + uvals[u] * extract(biv, lane0, bblk, u)
        return acc

    uiv0 = uidx_v[pl.ds(0, 16)]
    biv0 = bidx_v[pl.ds(0, 16)]
    fire(uembT_hbm, uiv0, 0, ua_v, sua)
    fire(bembT_hbm, biv0, 0, ba_v, sba)

    def step(k, carry):
        acc, uiv, biv = carry
        fire(uembT_hbm, uiv, 8, ub_v, sub)
        fire(bembT_hbm, biv, 8, bb_v, sbb)
        drain(ua_v, sua)
        drain(ba_v, sba)
        acc = mac(uiv, biv, 0, ua_v, ba_v, acc)
        nk = jnp.minimum(k + 1, NWAVE - 1)
        nuiv = uidx_v[pl.ds(nk * 16, 16)]
        nbiv = bidx_v[pl.ds(nk * 16, 16)]

        @pl.when(k + 1 < NWAVE)
        def _():
            fire(uembT_hbm, nuiv, 0, ua_v, sua)
            fire(bembT_hbm, nbiv, 0, ba_v, sba)

        drain(ub_v, sub)
        drain(bb_v, sbb)
        acc = mac(uiv, biv, 8, ub_v, bb_v, acc)
        return acc, nuiv, nbiv

    acc, _, _ = lax.fori_loop(
        0, NWAVE, step, (jnp.zeros((16,), jnp.float32), uiv0, biv0))
    acc_v[...] = acc
    pltpu.sync_copy(acc_v, partials_hbm.at[pl.ds(w * 16, 16)])


def _finish_body(idx2_hbm, partials_hbm, ubias_hbm, bbias_hbm, params_hbm,
                 out_hbm,
                 uidx_v, bidx_v, pt_v, ub_v, bb_v, params_v, out_v, sem):
    c = lax.axis_index("c")
    s = lax.axis_index("s")
    w = s * 2 + c
    base = w * W

    pltpu.sync_copy(idx2_hbm.at[0, pl.ds(base, W)], uidx_v)
    pltpu.sync_copy(idx2_hbm.at[1, pl.ds(base, W)], bidx_v)
    pltpu.sync_copy(partials_hbm, pt_v)
    pltpu.sync_copy(params_hbm, params_v)

    cp1 = pltpu.async_copy(ubias_hbm.at[uidx_v], ub_v, sem)
    cp2 = pltpu.async_copy(bbias_hbm.at[bidx_v], bb_v, sem)

    t = pt_v[pl.ds(0, 16)]
    for j in range(1, NT):
        t = t + pt_v[pl.ds(j * 16, 16)]
    S = t[0]
    for j in range(1, 16):
        S = S + t[j]

    scale = params_v[0, :]
    shift = params_v[1, :]
    cp1.wait()
    cp2.wait()

    def emit(j, carry):
        ch = pl.ds(j * 16, 16)
        x = (S + ub_v[ch] + bb_v[ch]) * scale + shift
        out_v[ch] = 1.0 / (1.0 + jnp.exp(-x))
        return carry

    lax.fori_loop(0, WCH, emit, 0)
    pltpu.sync_copy(out_v, out_hbm.at[pl.ds(base, W)])


@jax.jit
def _sc_call(idx2, uembT, ubias, bembT, bbias, params):
    mesh = plsc.VectorSubcoreMesh(core_axis_name="c", subcore_axis_name="s",
                                  num_cores=2)
    k1 = pl.kernel(
        _gather_body,
        out_type=jax.ShapeDtypeStruct((NT * 16,), jnp.float32),
        mesh=mesh,
        scratch_types=[
            pltpu.VMEM((W,), jnp.int32),
            pltpu.VMEM((W,), jnp.int32),
            pltpu.VMEM((8, 16, 128), jnp.float32),
            pltpu.VMEM((8, 16, 128), jnp.float32),
            pltpu.VMEM((8, 16, 128), jnp.float32),
            pltpu.VMEM((8, 16, 128), jnp.float32),
            pltpu.VMEM((16,), jnp.float32),
            pltpu.SemaphoreType.DMA,
            pltpu.SemaphoreType.DMA,
            pltpu.SemaphoreType.DMA,
            pltpu.SemaphoreType.DMA,
        ],
        compiler_params=pltpu.CompilerParams(use_tc_tiling_on_sc=True,
                                             needs_layout_passes=False,
                                             disable_bounds_checks=True),
    )
    partials = k1(idx2, uembT, bembT)

    k2 = pl.kernel(
        _finish_body,
        out_type=jax.ShapeDtypeStruct((BATCH,), jnp.float32),
        mesh=mesh,
        scratch_types=[
            pltpu.VMEM((W,), jnp.int32),
            pltpu.VMEM((W,), jnp.int32),
            pltpu.VMEM((NT * 16,), jnp.float32),
            pltpu.VMEM((W,), jnp.float32),
            pltpu.VMEM((W,), jnp.float32),
            pltpu.VMEM((2, 16), jnp.float32),
            pltpu.VMEM((W,), jnp.float32),
            pltpu.SemaphoreType.DMA,
        ],
        compiler_params=pltpu.CompilerParams(use_tc_tiling_on_sc=False),
    )
    return k2(idx2, partials, ubias, bbias, params)


def kernel(inputs, user_emb, user_bias, book_emb, book_bias,
           bn_gamma, bn_beta, bn_mean, bn_var):
    idx2 = inputs.T
    ubias = user_bias.reshape(-1)
    bbias = book_bias.reshape(-1)
    scale = bn_gamma * lax.rsqrt(bn_var + BN_EPS)
    shift = bn_beta - bn_mean * scale
    params = jnp.stack([jnp.broadcast_to(scale, (16,)),
                        jnp.broadcast_to(shift, (16,))])
    out = _sc_call(idx2, user_emb.T, ubias, book_emb.T, bbias, params)
    return out.reshape(BATCH, 1)
